# baseline (speedup 1.0000x reference)
import functools

import jax
import jax.numpy as jnp
from jax import lax
from jax.experimental import pallas as pl
from jax.experimental.pallas import tpu as pltpu
from jax.experimental.pallas import tpu_sc as plsc

_N = 10000
_E = 160000
_D = 256
_H = 256
_C = 64

_NC, _NS = 2, 16
_BLK = 128
_NPAD = 10240
_RPT = _NPAD // _NS
_EPAD = 163840
_NB16 = _EPAD // (_NS * _BLK)
_NB32 = _EPAD // (_NC * _NS * _BLK)

_MB = 256
_GRID = _NPAD // _MB

_sc_mesh = plsc.VectorSubcoreMesh(core_axis_name="c", subcore_axis_name="s")


def _fill_const(ref, rows, cols, value):
    @pl.loop(0, rows)
    def _(i):
        for cc in range(cols // 16):
            ref[i, pl.ds(cc * 16, 16)] = jnp.full((16,), value, jnp.float32)


def _deg_body(dst_hbm, degp_hbm, idx_v, ones_v, zero_v, acc_sh):
    cid = lax.axis_index("c")
    sid = lax.axis_index("s")
    _fill_const(ones_v, _BLK, 16, 1.0)
    _fill_const(zero_v, _BLK, 16, 0.0)

    @pl.loop(0, _RPT // _BLK)
    def _(k):
        pltpu.sync_copy(zero_v, acc_sh.at[pl.ds(sid * _RPT + k * _BLK, _BLK)])

    plsc.subcore_barrier()
    w = sid * _NC + cid
    pltpu.sync_copy(dst_hbm.at[w], idx_v)

    @pl.loop(0, _NB32)
    def _(j):
        pltpu.sync_copy(ones_v, acc_sh.at[idx_v.at[j]], add=True)

    plsc.subcore_barrier()
    pltpu.sync_copy(acc_sh.at[pl.ds(sid * _RPT, _RPT)],
                    degp_hbm.at[cid].at[pl.ds(sid * _RPT, _RPT)])


_deg_kernel = pl.kernel(
    _deg_body,
    out_type=jax.ShapeDtypeStruct((_NC, _NPAD, 16), jnp.float32),
    mesh=_sc_mesh,
    scratch_types=[
        pltpu.VMEM((_NB32, _BLK), jnp.int32),
        pltpu.VMEM((_BLK, 16), jnp.float32),
        pltpu.VMEM((_BLK, 16), jnp.float32),
        pltpu.VMEM_SHARED((_NPAD, 16), jnp.float32),
    ],
    compiler_params=pltpu.CompilerParams(use_tc_tiling_on_sc=False),
)


def _make_agg(cols, blk, ring, look):
    nblocks = _EPAD // (_NS * blk)
    nbt = nblocks // ring

    def body(g0_hbm, g1_hbm, src_hbm, dst_hbm, out_hbm,
             isrc_v, idst_v, zero_v, *rest):
        gbufs = rest[:ring]
        sems = rest[ring:2 * ring]
        acc_sh = rest[2 * ring]
        cid = lax.axis_index("c")
        sid = lax.axis_index("s")
        _fill_const(zero_v, 16, cols, 0.0)

        @pl.loop(0, _RPT // 16)
        def _(k):
            pltpu.sync_copy(zero_v, acc_sh.at[pl.ds(sid * _RPT + k * 16, 16)])

        plsc.subcore_barrier()
        pltpu.sync_copy(src_hbm.at[sid], isrc_v)
        pltpu.sync_copy(dst_hbm.at[sid], idst_v)

        def start_gather(j, b):
            @pl.when(cid == 0)
            def _():
                pltpu.async_copy(g0_hbm.at[isrc_v.at[j]], gbufs[b], sems[b])

            @pl.when(cid == 1)
            def _():
                pltpu.async_copy(g1_hbm.at[isrc_v.at[j]], gbufs[b], sems[b])

        def drain(b):
            pltpu.make_async_copy(g0_hbm.at[pl.ds(0, blk)],
                                  gbufs[b], sems[b]).wait()

        for b in range(look):
            start_gather(b, b)

        @pl.loop(0, nbt)
        def _(t):
            for r in range(ring):
                j = ring * t + r
                drain(r)
                pltpu.async_copy(gbufs[r], acc_sh.at[idst_v.at[j]],
                                 sems[r], add=True)
                b2 = (r + look) % ring
                if r < ring - look:
                    @pl.when(t >= 1)
                    def _():
                        drain(b2)

                    start_gather(j + look, b2)
                else:
                    drain(b2)

                    @pl.when(t < nbt - 1)
                    def _():
                        start_gather(j + look, b2)

        for b in range(look, ring):
            drain(b)
        plsc.subcore_barrier()
        pltpu.sync_copy(acc_sh.at[pl.ds(sid * _RPT, _RPT)],
                        out_hbm.at[cid].at[pl.ds(sid * _RPT, _RPT)])

    return pl.kernel(
        body,
        out_type=jax.ShapeDtypeStruct((_NC, _NPAD, cols), jnp.float32),
        mesh=_sc_mesh,
        scratch_types=(
            [pltpu.VMEM((nblocks, blk), jnp.int32),
             pltpu.VMEM((nblocks, blk), jnp.int32),
             pltpu.VMEM((16, cols), jnp.float32)]
            + [pltpu.VMEM((blk, cols), jnp.float32) for _ in range(ring)]
            + [pltpu.SemaphoreType.DMA for _ in range(ring)]
            + [pltpu.VMEM_SHARED((_NPAD, cols), jnp.float32)]
        ),
        compiler_params=pltpu.CompilerParams(use_tc_tiling_on_sc=False),
    )


_ABLK = 128
_agg64_kernel = _make_agg(64, _ABLK, ring=8, look=4)
_agg32_kernel = _make_agg(_C // 2, _ABLK, ring=8, look=4)


def _dis_of(degp_ref):
    deg = degp_ref[0, :, 0] + degp_ref[1, :, 0] + 1.0
    return lax.rsqrt(deg)


def _k2a_body(x_ref, w1_ref, h_ref):
    h_ref[...] = jnp.dot(x_ref[...], w1_ref[...],
                         preferred_element_type=jnp.float32)


def _k2b_body(h_ref, degp_ref, q0_ref, q1_ref, q2_ref, q3_ref):
    dis = _dis_of(degp_ref)
    g = h_ref[...] * dis[:, None]
    q0_ref[...] = g[:, 0:64]
    q1_ref[...] = g[:, 64:128]
    q2_ref[...] = g[:, 128:192]
    q3_ref[...] = g[:, 192:256]


def _k4_body(acca_ref, accb_ref, q0_ref, q1_ref, q2_ref, q3_ref, degp_ref,
             b1_ref, w2_ref, g2lo_ref, g2hi_ref):
    dis = _dis_of(degp_ref)
    s = jnp.concatenate(
        [acca_ref[0] + q0_ref[...], accb_ref[0] + q1_ref[...],
         acca_ref[1] + q2_ref[...], accb_ref[1] + q3_ref[...]], axis=1)
    x1 = jnp.maximum(s * dis[:, None] + b1_ref[...], 0.0)
    g2 = jnp.dot(x1, w2_ref[...], preferred_element_type=jnp.float32)
    g2 = g2 * dis[:, None]
    g2lo_ref[...] = g2[:, :_C // 2]
    g2hi_ref[...] = g2[:, _C // 2:]


def _k6_body(acc_ref, g2lo_ref, g2hi_ref, degp_ref, b2_ref, out_ref):
    dis = _dis_of(degp_ref)
    lo = acc_ref[0] + g2lo_ref[...]
    hi = acc_ref[1] + g2hi_ref[...]
    z = jnp.concatenate([lo, hi], axis=1) * dis[:, None] + b2_ref[...]
    m = jnp.max(z, axis=1, keepdims=True)
    lse = jnp.log(jnp.sum(jnp.exp(z - m), axis=1, keepdims=True)) + m
    out_ref[...] = z - lse


_degp_spec = pl.BlockSpec((_NC, _MB, 16), lambda i: (0, i, 0))

_q_spec = pl.BlockSpec((_MB, 64), lambda i: (i, 0))
_q_shape = jax.ShapeDtypeStruct((_NPAD, 64), jnp.float32)

_k2a_call = pl.pallas_call(
    _k2a_body,
    grid=(_GRID,),
    in_specs=[
        pl.BlockSpec((_MB, _D), lambda i: (i, 0)),
        pl.BlockSpec((_D, _H), lambda i: (0, 0)),
    ],
    out_specs=pl.BlockSpec((_MB, _H), lambda i: (i, 0)),
    out_shape=jax.ShapeDtypeStruct((_NPAD, _H), jnp.float32),
)

_k2b_call = pl.pallas_call(
    _k2b_body,
    grid=(_GRID,),
    in_specs=[
        pl.BlockSpec((_MB, _H), lambda i: (i, 0)),
        _degp_spec,
    ],
    out_specs=[_q_spec, _q_spec, _q_spec, _q_spec],
    out_shape=[_q_shape, _q_shape, _q_shape, _q_shape],
)

_k4_call = pl.pallas_call(
    _k4_body,
    grid=(_GRID,),
    in_specs=[
        pl.BlockSpec((_NC, _MB, 64), lambda i: (0, i, 0)),
        pl.BlockSpec((_NC, _MB, 64), lambda i: (0, i, 0)),
        _q_spec,
        _q_spec,
        _q_spec,
        _q_spec,
        _degp_spec,
        pl.BlockSpec((1, _H), lambda i: (0, 0)),
        pl.BlockSpec((_H, _C), lambda i: (0, 0)),
    ],
    out_specs=[
        pl.BlockSpec((_MB, _C // 2), lambda i: (i, 0)),
        pl.BlockSpec((_MB, _C // 2), lambda i: (i, 0)),
    ],
    out_shape=[
        jax.ShapeDtypeStruct((_NPAD, _C // 2), jnp.float32),
        jax.ShapeDtypeStruct((_NPAD, _C // 2), jnp.float32),
    ],
)

_k6_call = pl.pallas_call(
    _k6_body,
    grid=(_GRID,),
    in_specs=[
        pl.BlockSpec((_NC, _MB, _C // 2), lambda i: (0, i, 0)),
        pl.BlockSpec((_MB, _C // 2), lambda i: (i, 0)),
        pl.BlockSpec((_MB, _C // 2), lambda i: (i, 0)),
        _degp_spec,
        pl.BlockSpec((1, _C), lambda i: (0, 0)),
    ],
    out_specs=pl.BlockSpec((_MB, _C), lambda i: (i, 0)),
    out_shape=jax.ShapeDtypeStruct((_NPAD, _C), jnp.float32),
)


def kernel(features, edge_index, train_mask, W1, b1, W2, b2):
    src = edge_index[0]
    dst = edge_index[1]
    pad = _EPAD - src.shape[0]
    fill = jnp.full((pad,), _N, jnp.int32)
    srcp = jnp.concatenate([src.astype(jnp.int32), fill])
    dstp = jnp.concatenate([dst.astype(jnp.int32), fill])
    src16 = srcp.reshape(_NS, -1, _ABLK)
    dst16 = dstp.reshape(_NS, -1, _ABLK)
    src32 = srcp.reshape(_NC * _NS, _NB32, _BLK)
    dst32 = dstp.reshape(_NC * _NS, _NB32, _BLK)
    xp = jnp.pad(features, ((0, _NPAD - _N), (0, 0)))

    degp = _deg_kernel(dst32)
    h1 = _k2a_call(xp, W1)
    q0, q1, q2, q3 = _k2b_call(h1, degp)
    acca = _agg64_kernel(q0, q2, src16, dst16)
    accb = _agg64_kernel(q1, q3, src16, dst16)
    g2lo, g2hi = _k4_call(acca, accb, q0, q1, q2, q3, degp,
                          b1.reshape(1, _H), W2)
    acc2 = _agg32_kernel(g2lo, g2hi, src16, dst16)
    out = _k6_call(acc2, g2lo, g2hi, degp,
                   b2.reshape(1, _C))
    return out[:_N]


# speedup vs baseline: 1.0739x; 1.0739x over previous
import functools

import jax
import jax.numpy as jnp
from jax import lax
from jax.experimental import pallas as pl
from jax.experimental.pallas import tpu as pltpu
from jax.experimental.pallas import tpu_sc as plsc

_N = 10000
_E = 160000
_D = 256
_H = 256
_C = 64

_NC, _NS = 2, 16
_BLK = 128
_NPAD = 10240
_RPT = _NPAD // _NS
_EPAD = 163840
_NB16 = _EPAD // (_NS * _BLK)
_NB32 = _EPAD // (_NC * _NS * _BLK)

_MB = 256
_GRID = _NPAD // _MB

_sc_mesh = plsc.VectorSubcoreMesh(core_axis_name="c", subcore_axis_name="s")


def _fill_const(ref, rows, cols, value):
    @pl.loop(0, rows)
    def _(i):
        for cc in range(cols // 16):
            ref[i, pl.ds(cc * 16, 16)] = jnp.full((16,), value, jnp.float32)


def _deg_body(dst_hbm, degp_hbm, idx_v, ones_v, zero_v, acc_sh):
    cid = lax.axis_index("c")
    sid = lax.axis_index("s")
    _fill_const(ones_v, _BLK, 16, 1.0)
    _fill_const(zero_v, _BLK, 16, 0.0)

    @pl.loop(0, _RPT // _BLK)
    def _(k):
        pltpu.sync_copy(zero_v, acc_sh.at[pl.ds(sid * _RPT + k * _BLK, _BLK)])

    plsc.subcore_barrier()
    w = sid * _NC + cid
    pltpu.sync_copy(dst_hbm.at[w], idx_v)

    @pl.loop(0, _NB32)
    def _(j):
        pltpu.sync_copy(ones_v, acc_sh.at[idx_v.at[j]], add=True)

    plsc.subcore_barrier()
    pltpu.sync_copy(acc_sh.at[pl.ds(sid * _RPT, _RPT)],
                    degp_hbm.at[cid].at[pl.ds(sid * _RPT, _RPT)])


_deg_kernel = pl.kernel(
    _deg_body,
    out_type=jax.ShapeDtypeStruct((_NC, _NPAD, 16), jnp.float32),
    mesh=_sc_mesh,
    scratch_types=[
        pltpu.VMEM((_NB32, _BLK), jnp.int32),
        pltpu.VMEM((_BLK, 16), jnp.float32),
        pltpu.VMEM((_BLK, 16), jnp.float32),
        pltpu.VMEM_SHARED((_NPAD, 16), jnp.float32),
    ],
    compiler_params=pltpu.CompilerParams(use_tc_tiling_on_sc=False),
)


def _make_agg(cols, blk, ring, look):
    nblocks = _EPAD // (_NS * blk)
    nbt = nblocks // ring

    def body(g0_hbm, g1_hbm, src_hbm, dst_hbm, out_hbm,
             isrc_v, idst_v, zero_v, *rest):
        gbufs = rest[:ring]
        sems = rest[ring:2 * ring]
        acc_sh = rest[2 * ring]
        cid = lax.axis_index("c")
        sid = lax.axis_index("s")
        _fill_const(zero_v, 16, cols, 0.0)

        @pl.loop(0, _RPT // 16)
        def _(k):
            pltpu.sync_copy(zero_v, acc_sh.at[pl.ds(sid * _RPT + k * 16, 16)])

        plsc.subcore_barrier()
        pltpu.sync_copy(src_hbm.at[sid], isrc_v)
        pltpu.sync_copy(dst_hbm.at[sid], idst_v)

        def start_gather(j, b):
            @pl.when(cid == 0)
            def _():
                pltpu.async_copy(g0_hbm.at[isrc_v.at[j]], gbufs[b], sems[b])

            @pl.when(cid == 1)
            def _():
                pltpu.async_copy(g1_hbm.at[isrc_v.at[j]], gbufs[b], sems[b])

        def drain(b):
            pltpu.make_async_copy(g0_hbm.at[pl.ds(0, blk)],
                                  gbufs[b], sems[b]).wait()

        for b in range(look):
            start_gather(b, b)

        @pl.loop(0, nbt)
        def _(t):
            for r in range(ring):
                j = ring * t + r
                drain(r)
                pltpu.async_copy(gbufs[r], acc_sh.at[idst_v.at[j]],
                                 sems[r], add=True)
                b2 = (r + look) % ring
                if r < ring - look:
                    @pl.when(t >= 1)
                    def _():
                        drain(b2)

                    start_gather(j + look, b2)
                else:
                    drain(b2)

                    @pl.when(t < nbt - 1)
                    def _():
                        start_gather(j + look, b2)

        for b in range(look, ring):
            drain(b)
        plsc.subcore_barrier()
        pltpu.sync_copy(acc_sh.at[pl.ds(sid * _RPT, _RPT)],
                        out_hbm.at[cid].at[pl.ds(sid * _RPT, _RPT)])

    return pl.kernel(
        body,
        out_type=jax.ShapeDtypeStruct((_NC, _NPAD, cols), jnp.float32),
        mesh=_sc_mesh,
        scratch_types=(
            [pltpu.VMEM((nblocks, blk), jnp.int32),
             pltpu.VMEM((nblocks, blk), jnp.int32),
             pltpu.VMEM((16, cols), jnp.float32)]
            + [pltpu.VMEM((blk, cols), jnp.float32) for _ in range(ring)]
            + [pltpu.SemaphoreType.DMA for _ in range(ring)]
            + [pltpu.VMEM_SHARED((_NPAD, cols), jnp.float32)]
        ),
        compiler_params=pltpu.CompilerParams(use_tc_tiling_on_sc=False),
    )


_ABLK = 128
_agg64_kernel = _make_agg(64, _ABLK, ring=8, look=4)
_agg32_kernel = _make_agg(_C // 2, _ABLK, ring=8, look=4)


def _dis_of(degp_ref):
    deg = degp_ref[0, :, 0] + degp_ref[1, :, 0] + 1.0
    return lax.rsqrt(deg)


def _k2_body(x_ref, degp_ref, w1_ref, q0_ref, q1_ref, q2_ref, q3_ref):
    dis = _dis_of(degp_ref)
    h = jnp.dot(x_ref[...], w1_ref[...], preferred_element_type=jnp.float32)
    g = h * dis[:, None]
    q0_ref[...] = g[:, 0:64]
    q1_ref[...] = g[:, 64:128]
    q2_ref[...] = g[:, 128:192]
    q3_ref[...] = g[:, 192:256]


def _k4_body(acca_ref, accb_ref, q0_ref, q1_ref, q2_ref, q3_ref, degp_ref,
             b1_ref, w2_ref, g2lo_ref, g2hi_ref):
    dis = _dis_of(degp_ref)
    s = jnp.concatenate(
        [acca_ref[0] + q0_ref[...], accb_ref[0] + q1_ref[...],
         acca_ref[1] + q2_ref[...], accb_ref[1] + q3_ref[...]], axis=1)
    x1 = jnp.maximum(s * dis[:, None] + b1_ref[...], 0.0)
    g2 = jnp.dot(x1, w2_ref[...], preferred_element_type=jnp.float32)
    g2 = g2 * dis[:, None]
    g2lo_ref[...] = g2[:, :_C // 2]
    g2hi_ref[...] = g2[:, _C // 2:]


def _k6_body(acc_ref, g2lo_ref, g2hi_ref, degp_ref, b2_ref, out_ref):
    dis = _dis_of(degp_ref)
    lo = acc_ref[0] + g2lo_ref[...]
    hi = acc_ref[1] + g2hi_ref[...]
    z = jnp.concatenate([lo, hi], axis=1) * dis[:, None] + b2_ref[...]
    m = jnp.max(z, axis=1, keepdims=True)
    lse = jnp.log(jnp.sum(jnp.exp(z - m), axis=1, keepdims=True)) + m
    out_ref[...] = z - lse


_degp_spec = pl.BlockSpec((_NC, _MB, 16), lambda i: (0, i, 0))

_q_spec = pl.BlockSpec((_MB, 64), lambda i: (i, 0))
_q_shape = jax.ShapeDtypeStruct((_NPAD, 64), jnp.float32)

_k2_call = pl.pallas_call(
    _k2_body,
    grid=(_GRID,),
    in_specs=[
        pl.BlockSpec((_MB, _D), lambda i: (i, 0)),
        _degp_spec,
        pl.BlockSpec((_D, _H), lambda i: (0, 0)),
    ],
    out_specs=[_q_spec, _q_spec, _q_spec, _q_spec],
    out_shape=[_q_shape, _q_shape, _q_shape, _q_shape],
)

_k4_call = pl.pallas_call(
    _k4_body,
    grid=(_GRID,),
    in_specs=[
        pl.BlockSpec((_NC, _MB, 64), lambda i: (0, i, 0)),
        pl.BlockSpec((_NC, _MB, 64), lambda i: (0, i, 0)),
        _q_spec,
        _q_spec,
        _q_spec,
        _q_spec,
        _degp_spec,
        pl.BlockSpec((1, _H), lambda i: (0, 0)),
        pl.BlockSpec((_H, _C), lambda i: (0, 0)),
    ],
    out_specs=[
        pl.BlockSpec((_MB, _C // 2), lambda i: (i, 0)),
        pl.BlockSpec((_MB, _C // 2), lambda i: (i, 0)),
    ],
    out_shape=[
        jax.ShapeDtypeStruct((_NPAD, _C // 2), jnp.float32),
        jax.ShapeDtypeStruct((_NPAD, _C // 2), jnp.float32),
    ],
)

_k6_call = pl.pallas_call(
    _k6_body,
    grid=(_GRID,),
    in_specs=[
        pl.BlockSpec((_NC, _MB, _C // 2), lambda i: (0, i, 0)),
        pl.BlockSpec((_MB, _C // 2), lambda i: (i, 0)),
        pl.BlockSpec((_MB, _C // 2), lambda i: (i, 0)),
        _degp_spec,
        pl.BlockSpec((1, _C), lambda i: (0, 0)),
    ],
    out_specs=pl.BlockSpec((_MB, _C), lambda i: (i, 0)),
    out_shape=jax.ShapeDtypeStruct((_NPAD, _C), jnp.float32),
)


def kernel(features, edge_index, train_mask, W1, b1, W2, b2):
    src = edge_index[0]
    dst = edge_index[1]
    pad = _EPAD - src.shape[0]
    fill = jnp.full((pad,), _N, jnp.int32)
    srcp = jnp.concatenate([src.astype(jnp.int32), fill])
    dstp = jnp.concatenate([dst.astype(jnp.int32), fill])
    src16 = srcp.reshape(_NS, -1, _ABLK)
    dst16 = dstp.reshape(_NS, -1, _ABLK)
    src32 = srcp.reshape(_NC * _NS, _NB32, _BLK)
    dst32 = dstp.reshape(_NC * _NS, _NB32, _BLK)
    xp = jnp.pad(features, ((0, _NPAD - _N), (0, 0)))

    degp = _deg_kernel(dst32)
    q0, q1, q2, q3 = _k2_call(xp, degp, W1)
    acca = _agg64_kernel(q0, q2, src16, dst16)
    accb = _agg64_kernel(q1, q3, src16, dst16)
    g2lo, g2hi = _k4_call(acca, accb, q0, q1, q2, q3, degp,
                          b1.reshape(1, _H), W2)
    acc2 = _agg32_kernel(g2lo, g2hi, src16, dst16)
    out = _k6_call(acc2, g2lo, g2hi, degp,
                   b2.reshape(1, _C))
    return out[:_N]


# speedup vs baseline: 1.0744x; 1.0005x over previous
import jax
import jax.numpy as jnp
from jax import lax
from jax.experimental import pallas as pl
from jax.experimental.pallas import tpu as pltpu
from jax.experimental.pallas import tpu_sc as plsc

_N = 10000
_E = 160000
_D = 256
_H = 256
_C = 64

_NC, _NS = 2, 16
_BLK = 128
_NPAD = 10240
_RPT = _NPAD // _NS
_EPAD = 163840
_NB32 = _EPAD // (_NC * _NS * _BLK)

_MB = 256
_GRID = _NPAD // _MB

_sc_mesh = plsc.VectorSubcoreMesh(core_axis_name="c", subcore_axis_name="s")


def _fill_const(ref, rows, cols, value):
    @pl.loop(0, rows)
    def _(i):
        for cc in range(cols // 16):
            ref[i, pl.ds(cc * 16, 16)] = jnp.full((16,), value, jnp.float32)


def _deg_body(dst_hbm, degp_hbm, idx_v, ones_v, zero_v, acc_sh):
    cid = lax.axis_index("c")
    sid = lax.axis_index("s")
    _fill_const(ones_v, _BLK, 16, 1.0)
    _fill_const(zero_v, _BLK, 16, 0.0)

    @pl.loop(0, _RPT // _BLK)
    def _(k):
        pltpu.sync_copy(zero_v, acc_sh.at[pl.ds(sid * _RPT + k * _BLK, _BLK)])

    plsc.subcore_barrier()
    w = sid * _NC + cid
    pltpu.sync_copy(dst_hbm.at[w], idx_v)

    @pl.loop(0, _NB32)
    def _(j):
        pltpu.sync_copy(ones_v, acc_sh.at[idx_v.at[j]], add=True)

    plsc.subcore_barrier()
    pltpu.sync_copy(acc_sh.at[pl.ds(sid * _RPT, _RPT)],
                    degp_hbm.at[cid].at[pl.ds(sid * _RPT, _RPT)])


_deg_kernel = pl.kernel(
    _deg_body,
    out_type=jax.ShapeDtypeStruct((_NC, _NPAD, 16), jnp.float32),
    mesh=_sc_mesh,
    scratch_types=[
        pltpu.VMEM((_NB32, _BLK), jnp.int32),
        pltpu.VMEM((_BLK, 16), jnp.float32),
        pltpu.VMEM((_BLK, 16), jnp.float32),
        pltpu.VMEM_SHARED((_NPAD, 16), jnp.float32),
    ],
    compiler_params=pltpu.CompilerParams(use_tc_tiling_on_sc=False),
)


def _make_agg(cols, blk, ring, look):
    nblocks = _EPAD // (_NS * blk)
    nbt = nblocks // ring

    def body(g0_hbm, g1_hbm, src_hbm, dst_hbm, out_hbm,
             isrc_v, idst_v, zero_v, *rest):
        gbufs = rest[:ring]
        sems = rest[ring:2 * ring]
        acc_sh = rest[2 * ring]
        cid = lax.axis_index("c")
        sid = lax.axis_index("s")
        _fill_const(zero_v, 16, cols, 0.0)

        @pl.loop(0, _RPT // 16)
        def _(k):
            pltpu.sync_copy(zero_v, acc_sh.at[pl.ds(sid * _RPT + k * 16, 16)])

        plsc.subcore_barrier()
        pltpu.sync_copy(src_hbm.at[sid], isrc_v)
        pltpu.sync_copy(dst_hbm.at[sid], idst_v)

        def start_gather(j, b):
            @pl.when(cid == 0)
            def _():
                pltpu.async_copy(g0_hbm.at[isrc_v.at[j]], gbufs[b], sems[b])

            @pl.when(cid == 1)
            def _():
                pltpu.async_copy(g1_hbm.at[isrc_v.at[j]], gbufs[b], sems[b])

        def drain(b):
            pltpu.make_async_copy(g0_hbm.at[pl.ds(0, blk)],
                                  gbufs[b], sems[b]).wait()

        for b in range(look):
            start_gather(b, b)

        @pl.loop(0, nbt)
        def _(t):
            for r in range(ring):
                j = ring * t + r
                drain(r)
                pltpu.async_copy(gbufs[r], acc_sh.at[idst_v.at[j]],
                                 sems[r], add=True)
                b2 = (r + look) % ring
                if r < ring - look:
                    @pl.when(t >= 1)
                    def _():
                        drain(b2)

                    start_gather(j + look, b2)
                else:
                    drain(b2)

                    @pl.when(t < nbt - 1)
                    def _():
                        start_gather(j + look, b2)

        for b in range(look, ring):
            drain(b)
        plsc.subcore_barrier()
        pltpu.sync_copy(acc_sh.at[pl.ds(sid * _RPT, _RPT)],
                        out_hbm.at[cid].at[pl.ds(sid * _RPT, _RPT)])

    return pl.kernel(
        body,
        out_type=jax.ShapeDtypeStruct((_NC, _NPAD, cols), jnp.float32),
        mesh=_sc_mesh,
        scratch_types=(
            [pltpu.VMEM((nblocks, blk), jnp.int32),
             pltpu.VMEM((nblocks, blk), jnp.int32),
             pltpu.VMEM((16, cols), jnp.float32)]
            + [pltpu.VMEM((blk, cols), jnp.float32) for _ in range(ring)]
            + [pltpu.SemaphoreType.DMA for _ in range(ring)]
            + [pltpu.VMEM_SHARED((_NPAD, cols), jnp.float32)]
        ),
        compiler_params=pltpu.CompilerParams(use_tc_tiling_on_sc=False),
    )


_ABLK = 128
_agg64_kernel = _make_agg(64, _ABLK, ring=8, look=4)
_agg32_kernel = _make_agg(_C // 2, _ABLK, ring=8, look=4)


def _dis_of(degp_ref):
    deg = degp_ref[0, :, 0] + degp_ref[1, :, 0] + 1.0
    return lax.rsqrt(deg)


def _k2_body(x_ref, degp_ref, w1_ref, q0_ref, q1_ref, q2_ref, q3_ref):
    dis = _dis_of(degp_ref)
    h = jnp.dot(x_ref[...], w1_ref[...], preferred_element_type=jnp.float32)
    g = h * dis[:, None]
    q0_ref[...] = g[:, 0:64]
    q1_ref[...] = g[:, 64:128]
    q2_ref[...] = g[:, 128:192]
    q3_ref[...] = g[:, 192:256]


def _k4_body(acca_ref, accb_ref, q0_ref, q1_ref, q2_ref, q3_ref, degp_ref,
             b1_ref, w2_ref, g2lo_ref, g2hi_ref):
    dis = _dis_of(degp_ref)
    s = jnp.concatenate(
        [acca_ref[0] + q0_ref[...], accb_ref[0] + q1_ref[...],
         acca_ref[1] + q2_ref[...], accb_ref[1] + q3_ref[...]], axis=1)
    x1 = jnp.maximum(s * dis[:, None] + b1_ref[...], 0.0)
    g2 = jnp.dot(x1, w2_ref[...], preferred_element_type=jnp.float32)
    g2 = g2 * dis[:, None]
    g2lo_ref[...] = g2[:, :_C // 2]
    g2hi_ref[...] = g2[:, _C // 2:]


def _k6_body(acc_ref, g2lo_ref, g2hi_ref, degp_ref, b2_ref, out_ref):
    dis = _dis_of(degp_ref)
    lo = acc_ref[0] + g2lo_ref[...]
    hi = acc_ref[1] + g2hi_ref[...]
    z = jnp.concatenate([lo, hi], axis=1) * dis[:, None] + b2_ref[...]
    m = jnp.max(z, axis=1, keepdims=True)
    lse = jnp.log(jnp.sum(jnp.exp(z - m), axis=1, keepdims=True)) + m
    out_ref[...] = z - lse


_degp_spec = pl.BlockSpec((_NC, _MB, 16), lambda i: (0, i, 0))

_q_spec = pl.BlockSpec((_MB, 64), lambda i: (i, 0))
_q_shape = jax.ShapeDtypeStruct((_NPAD, 64), jnp.float32)

_k2_call = pl.pallas_call(
    _k2_body,
    grid=(_GRID,),
    in_specs=[
        pl.BlockSpec((_MB, _D), lambda i: (i, 0)),
        _degp_spec,
        pl.BlockSpec((_D, _H), lambda i: (0, 0)),
    ],
    out_specs=[_q_spec, _q_spec, _q_spec, _q_spec],
    out_shape=[_q_shape, _q_shape, _q_shape, _q_shape],
)

_k4_call = pl.pallas_call(
    _k4_body,
    grid=(_GRID,),
    in_specs=[
        pl.BlockSpec((_NC, _MB, 64), lambda i: (0, i, 0)),
        pl.BlockSpec((_NC, _MB, 64), lambda i: (0, i, 0)),
        _q_spec,
        _q_spec,
        _q_spec,
        _q_spec,
        _degp_spec,
        pl.BlockSpec((1, _H), lambda i: (0, 0)),
        pl.BlockSpec((_H, _C), lambda i: (0, 0)),
    ],
    out_specs=[
        pl.BlockSpec((_MB, _C // 2), lambda i: (i, 0)),
        pl.BlockSpec((_MB, _C // 2), lambda i: (i, 0)),
    ],
    out_shape=[
        jax.ShapeDtypeStruct((_NPAD, _C // 2), jnp.float32),
        jax.ShapeDtypeStruct((_NPAD, _C // 2), jnp.float32),
    ],
)

_k6_call = pl.pallas_call(
    _k6_body,
    grid=(_GRID,),
    in_specs=[
        pl.BlockSpec((_NC, _MB, _C // 2), lambda i: (0, i, 0)),
        pl.BlockSpec((_MB, _C // 2), lambda i: (i, 0)),
        pl.BlockSpec((_MB, _C // 2), lambda i: (i, 0)),
        _degp_spec,
        pl.BlockSpec((1, _C), lambda i: (0, 0)),
    ],
    out_specs=pl.BlockSpec((_MB, _C), lambda i: (i, 0)),
    out_shape=jax.ShapeDtypeStruct((_NPAD, _C), jnp.float32),
)


def kernel(features, edge_index, train_mask, W1, b1, W2, b2):
    src = edge_index[0]
    dst = edge_index[1]
    pad = _EPAD - src.shape[0]
    fill = jnp.full((pad,), _N, jnp.int32)
    srcp = jnp.concatenate([src.astype(jnp.int32), fill])
    dstp = jnp.concatenate([dst.astype(jnp.int32), fill])
    src16 = srcp.reshape(_NS, -1, _ABLK)
    dst16 = dstp.reshape(_NS, -1, _ABLK)
    src32 = srcp.reshape(_NC * _NS, _NB32, _BLK)
    dst32 = dstp.reshape(_NC * _NS, _NB32, _BLK)
    xp = jnp.pad(features, ((0, _NPAD - _N), (0, 0)))

    degp = _deg_kernel(dst32)
    q0, q1, q2, q3 = _k2_call(xp, degp, W1)
    acca = _agg64_kernel(q0, q2, src16, dst16)
    accb = _agg64_kernel(q1, q3, src16, dst16)
    g2lo, g2hi = _k4_call(acca, accb, q0, q1, q2, q3, degp,
                          b1.reshape(1, _H), W2)
    acc2 = _agg32_kernel(g2lo, g2hi, src16, dst16)
    out = _k6_call(acc2, g2lo, g2hi, degp,
                   b2.reshape(1, _C))
    return out[:_N]
